# baseline (speedup 1.0000x reference)
import functools

import jax
import jax.numpy as jnp
import numpy as np
from jax import lax
from jax.experimental import pallas as pl
from jax.experimental.pallas import tpu as pltpu
from jax.experimental.pallas import tpu_sc as plsc

B, N, C = 2, 20000, 91
CF = C - 1
CP = 96
R = B * CP
KW = 1024
PRE = 1000
POST = 100
NV = N // 16
NW = 32
RPW = R // NW
NB = KW // 128
BBOX_XFORM_CLIP = float(np.log(1000.0 / 16.0))
NEG_INF = float("-inf")


def _k1a_body(cls_ref, st_ref):
    x = cls_ref[0]
    m = jnp.max(x, axis=-1, keepdims=True)
    e = jnp.exp(x - m)
    s = jnp.sum(e, axis=-1, keepdims=True)
    p = e / s
    pf = p[:, 1:]
    pp = jnp.concatenate([pf, jnp.zeros((N, CP - CF), jnp.float32)], axis=1)
    st_ref[0] = pp.T


def _k1a(class_outputs):
    return pl.pallas_call(
        _k1a_body,
        grid=(B,),
        in_specs=[pl.BlockSpec((1, N, C), lambda b: (b, 0, 0))],
        out_specs=pl.BlockSpec((1, CP, N), lambda b: (b, 0, 0)),
        out_shape=jax.ShapeDtypeStruct((B, CP, N), jnp.float32),
    )(class_outputs)


def _k1b_body(st_ref, tb_ref, ngt_ref):
    s = st_ref[0]
    bits = lax.bitcast_convert_type(s, jnp.int32)

    def bs(_, lohi):
        lo, hi = lohi
        mid = lo + (hi - lo) // 2
        cnt = jnp.sum((bits > mid).astype(jnp.int32), axis=1, keepdims=True)
        take = cnt < PRE
        return (jnp.where(take, lo, mid + 1), jnp.where(take, mid, hi))

    lo0 = jnp.zeros((CP, 1), jnp.int32)
    hi0 = jnp.full((CP, 1), jnp.int32(0x3F800000))
    lo, _ = lax.fori_loop(0, 31, bs, (lo0, hi0))
    ngt = jnp.sum((bits > lo).astype(jnp.int32), axis=1, keepdims=True)
    tb_ref[0] = lo
    ngt_ref[0] = ngt


def _k1b(st):
    return pl.pallas_call(
        _k1b_body,
        grid=(B,),
        in_specs=[pl.BlockSpec((1, CP, N), lambda b: (b, 0, 0))],
        out_specs=[
            pl.BlockSpec((1, CP, 1), lambda b: (b, 0, 0)),
            pl.BlockSpec((1, CP, 1), lambda b: (b, 0, 0)),
        ],
        out_shape=[
            jax.ShapeDtypeStruct((B, CP, 1), jnp.int32),
            jax.ShapeDtypeStruct((B, CP, 1), jnp.int32),
        ],
    )(st)


def _k2_body(st_hbm, tb_hbm, eq_hbm, ss_hbm, si_hbm,
             row_v, sbuf, ibuf, ebuf, t_v, e_v, sem):
    core = lax.axis_index("c")
    sub = lax.axis_index("s")
    wid = sub * 2 + core
    for rr in range(RPW):
        r = wid * RPW + rr
        pltpu.sync_copy(st_hbm.at[r], row_v)
        pltpu.sync_copy(tb_hbm.at[r], t_v)
        pltpu.sync_copy(eq_hbm.at[r], e_v)
        tvec = t_v[...]
        eq0 = jnp.sum(e_v[...], axis=0)
        for t in range(62, KW // 16 + 1):
            sbuf[pl.ds(t * 16, 16)] = jnp.full((16,), NEG_INF, jnp.float32)
            ibuf[pl.ds(t * 16, 16)] = jnp.zeros((16,), jnp.int32)

        def body(jj, carry):
            ptr, eptr = carry
            sv = row_v[pl.ds(jj * 16, 16)]
            gt = sv > tvec
            eq = sv == tvec
            cnt = jnp.sum(gt.astype(jnp.int32), axis=0)
            ecnt = jnp.sum(eq.astype(jnp.int32), axis=0)
            idxv = lax.iota(jnp.int32, 16) + jj * 16
            plsc.store_compressed(sbuf.at[pl.ds(ptr, 16)], sv, mask=gt)
            plsc.store_compressed(ibuf.at[pl.ds(ptr, 16)], idxv, mask=gt)
            plsc.store_compressed(ebuf.at[pl.ds(eptr, 16)], idxv, mask=eq)
            return ptr + cnt, eptr + ecnt

        ngt, _ = lax.fori_loop(0, NV, body, (jnp.int32(0), jnp.int32(0)))

        def post(t, carry):
            base = t * 16
            lm = (lax.iota(jnp.int32, 16) + base) < eq0
            ev = ebuf[pl.ds(base, 16)]
            plsc.store_compressed(sbuf.at[pl.ds(ngt + base, 16)], tvec, mask=lm)
            plsc.store_compressed(ibuf.at[pl.ds(ngt + base, 16)], ev, mask=lm)
            return carry

        lax.fori_loop(0, (eq0 + 15) // 16, post, 0)
        pltpu.sync_copy(sbuf.at[pl.ds(0, KW)], ss_hbm.at[r])
        pltpu.sync_copy(ibuf.at[pl.ds(0, KW)], si_hbm.at[r])


def _k2(st, tb16, eq16):
    mesh = plsc.VectorSubcoreMesh(core_axis_name="c", subcore_axis_name="s")
    return pl.kernel(
        _k2_body,
        out_type=[
            jax.ShapeDtypeStruct((R, KW), jnp.float32),
            jax.ShapeDtypeStruct((R, KW), jnp.int32),
        ],
        mesh=mesh,
        compiler_params=pltpu.CompilerParams(needs_layout_passes=False),
        scratch_types=[
            pltpu.VMEM((N,), jnp.float32),
            pltpu.VMEM((KW + 16,), jnp.float32),
            pltpu.VMEM((KW + 16,), jnp.int32),
            pltpu.VMEM((N + 16,), jnp.int32),
            pltpu.VMEM((16,), jnp.float32),
            pltpu.VMEM((16,), jnp.int32),
            pltpu.SemaphoreType.DMA,
        ],
    )(st, tb16, eq16)


def _rl(x, j):
    return jnp.concatenate([x[:, j:], x[:, :j]], axis=1)


def _rr(x, j):
    return jnp.concatenate([x[:, KW - j:], x[:, :KW - j]], axis=1)


def _k3_body(s_ref, i_ref, so_ref, io_ref):
    s = s_ref[...]
    ii = i_ref[...]
    it = lax.broadcasted_iota(jnp.int32, (R, KW), 1)
    k = 2
    while k <= KW:
        j = k // 2
        while j >= 1:
            bitj = (it & j) != 0
            sp = jnp.where(bitj, _rr(s, j), _rl(s, j))
            ip = jnp.where(bitj, _rr(ii, j), _rl(ii, j))
            first = (s > sp) | ((s == sp) & (ii < ip))
            want_first = (~bitj) == ((it & k) == 0)
            take_self = first == want_first
            s = jnp.where(take_self, s, sp)
            ii = jnp.where(take_self, ii, ip)
            j //= 2
        k *= 2
    so_ref[...] = s
    io_ref[...] = ii


def _k3(s, i):
    return pl.pallas_call(
        _k3_body,
        out_shape=[
            jax.ShapeDtypeStruct((R, KW), jnp.float32),
            jax.ShapeDtypeStruct((R, KW), jnp.int32),
        ],
    )(s, i)


def _k4_body(si_hbm, boxflat_hbm, anchflat_hbm, hwh_hbm, hww_hbm,
             y1_hbm, x1_hbm, y2_hbm, x2_hbm,
             idx_v, eim, aim,
             eyb, exb, ehb, ewb, ay1b, ax1b, ay2b, ax2b,
             y1b, x1b, y2b, x2b, hwh_v, hww_v, sem):
    core = lax.axis_index("c")
    sub = lax.axis_index("s")
    wid = sub * 2 + core
    ebufs = [eyb, exb, ehb, ewb]
    abufs = [ay1b, ax1b, ay2b, ax2b]

    def row_fn(rr, _):
        r = wid * RPW + rr
        b = r // CP
        c = r % CP
        cls = jnp.minimum(c + 1, CF)
        base_e4 = (b * (N * C) + cls) * 4
        base_a4 = b * N * 4
        pltpu.sync_copy(si_hbm.at[r], idx_v)
        pltpu.sync_copy(hwh_hbm.at[r], hwh_v)
        pltpu.sync_copy(hww_hbm.at[r], hww_v)
        hv = hwh_v[...]
        wv = hww_v[...]

        def fill(jj, carry):
            v = idx_v[pl.ds(jj * 16, 16)]
            e4 = v * (C * 4) + base_e4
            a4 = v * 4 + base_a4
            ch = jj // 8
            off = (jj % 8) * 16
            for f in range(4):
                eim[f * 8 + ch, pl.ds(off, 16)] = e4 + f
                aim[f * 8 + ch, pl.ds(off, 16)] = a4 + f
            return carry

        lax.fori_loop(0, KW // 16, fill, 0)
        cps = []
        for f in range(4):
            for ch in range(KW // 128):
                cps.append(pltpu.async_copy(
                    boxflat_hbm.at[eim.at[f * 8 + ch]],
                    ebufs[f].at[pl.ds(ch * 128, 128)], sem))
                cps.append(pltpu.async_copy(
                    anchflat_hbm.at[aim.at[f * 8 + ch]],
                    abufs[f].at[pl.ds(ch * 128, 128)], sem))
        for cp in cps:
            cp.wait()

        def dec(jj, carry):
            sl = pl.ds(jj * 16, 16)
            ey = eyb[sl]
            ex = exb[sl]
            eh = ehb[sl]
            ew = ewb[sl]
            ay1 = ay1b[sl]
            ax1 = ax1b[sl]
            ay2 = ay2b[sl]
            ax2 = ax2b[sl]
            dy = ey / 10.0
            dx = ex / 10.0
            dh = jnp.minimum(eh / 5.0, BBOX_XFORM_CLIP)
            dw = jnp.minimum(ew / 5.0, BBOX_XFORM_CLIP)
            a_h = ay2 - ay1
            a_w = ax2 - ax1
            a_cy = ay1 + 0.5 * a_h
            a_cx = ax1 + 0.5 * a_w
            cy = dy * a_h + a_cy
            cx = dx * a_w + a_cx
            hh = jnp.exp(dh) * a_h
            ww = jnp.exp(dw) * a_w
            zf = jnp.zeros((16,), jnp.float32)
            y1b[sl] = jnp.minimum(jnp.maximum(cy - 0.5 * hh, zf), hv)
            x1b[sl] = jnp.minimum(jnp.maximum(cx - 0.5 * ww, zf), wv)
            y2b[sl] = jnp.minimum(jnp.maximum(cy + 0.5 * hh, zf), hv)
            x2b[sl] = jnp.minimum(jnp.maximum(cx + 0.5 * ww, zf), wv)
            return carry

        lax.fori_loop(0, KW // 16, dec, 0)
        pltpu.sync_copy(y1b, y1_hbm.at[r])
        pltpu.sync_copy(x1b, x1_hbm.at[r])
        pltpu.sync_copy(y2b, y2_hbm.at[r])
        pltpu.sync_copy(x2b, x2_hbm.at[r])
        return _

    lax.fori_loop(0, RPW, row_fn, 0)


def _k4(si, boxrows, anchrows, hwh, hww):
    mesh = plsc.VectorSubcoreMesh(core_axis_name="c", subcore_axis_name="s")
    plane = jax.ShapeDtypeStruct((R, KW), jnp.float32)
    return pl.kernel(
        _k4_body,
        out_type=[plane, plane, plane, plane],
        mesh=mesh,
        compiler_params=pltpu.CompilerParams(needs_layout_passes=False),
        scratch_types=(
            [pltpu.VMEM((KW,), jnp.int32),
             pltpu.VMEM((32, 128), jnp.int32),
             pltpu.VMEM((32, 128), jnp.int32)]
            + [pltpu.VMEM((KW,), jnp.float32) for _ in range(12)]
            + [pltpu.VMEM((16,), jnp.float32) for _ in range(2)]
            + [pltpu.SemaphoreType.DMA]
        ),
    )(si, boxrows, anchrows, hwh, hww)


RB = 8


def _k5a_body(y1_ref, x1_ref, y2_ref, x2_ref, ss_ref, mk_ref):
    y1 = y1_ref[...]; x1 = x1_ref[...]; y2 = y2_ref[...]; x2 = x2_ref[...]
    area = (y2 - y1) * (x2 - x1)
    kb = [jnp.ones((RB, 128), jnp.bool_) for _ in range(NB)]

    def _bi(a):
        return lax.broadcast_in_dim(a, (RB, 128, 128), (0, 1))

    def _bj(a):
        return lax.broadcast_in_dim(a, (RB, 128, 128), (0, 2))

    def iou_blk(ai, bj):
        sa = slice(ai * 128, (ai + 1) * 128)
        sb = slice(bj * 128, (bj + 1) * 128)
        ih = jnp.maximum(
            jnp.minimum(_bi(y2[:, sa]), _bj(y2[:, sb]))
            - jnp.maximum(_bi(y1[:, sa]), _bj(y1[:, sb])), 0.0)
        iw = jnp.maximum(
            jnp.minimum(_bi(x2[:, sa]), _bj(x2[:, sb]))
            - jnp.maximum(_bi(x1[:, sa]), _bj(x1[:, sb])), 0.0)
        inter = ih * iw
        denom = _bi(area[:, sa]) + _bj(area[:, sb]) - inter + 1e-8
        return inter / denom > 0.5

    tri_t = (lax.broadcasted_iota(jnp.int32, (1, 128, 128), 1)
             > lax.broadcasted_iota(jnp.int32, (1, 128, 128), 2))
    for bi in range(NB):
        M = iou_blk(bi, bi) & tri_t
        pre = kb[bi]

        def _bjm(a_i32):
            return _bj(a_i32) != 0

        def fx_body(carry):
            alive_i, _ = carry
            supp = jnp.any(M & _bjm(alive_i), axis=2)
            new = pre & (~supp)
            new_i = new.astype(jnp.int32)
            return (new_i, jnp.any(new_i != alive_i))

        alive_i, _ = lax.while_loop(
            lambda cr: cr[1], fx_body, (pre.astype(jnp.int32), jnp.bool_(True)))
        alive = alive_i != 0
        kb[bi] = alive
        for bj in range(bi + 1, NB):
            Mc = iou_blk(bj, bi)
            supp = jnp.any(Mc & _bjm(alive_i), axis=2)
            kb[bj] = kb[bj] & (~supp)

    keep = jnp.concatenate(kb, axis=1)
    ss = ss_ref[...]
    masked = jnp.where(keep, ss, -1.0)
    colit = lax.broadcasted_iota(jnp.int32, (RB, KW), 1)
    rowit = (lax.broadcasted_iota(jnp.int32, (RB, KW), 0)
             + pl.program_id(0) * RB)
    valid = (colit < PRE) & ((rowit % CP) < CF)
    mk_ref[...] = jnp.where(valid, masked, NEG_INF)


def _k5a(y1, x1, y2, x2, ss):
    bs = pl.BlockSpec((RB, KW), lambda g: (g, 0))
    return pl.pallas_call(
        _k5a_body,
        grid=(R // RB,),
        in_specs=[bs, bs, bs, bs, bs],
        out_specs=bs,
        out_shape=jax.ShapeDtypeStruct((R, KW), jnp.float32),
    )(y1, x1, y2, x2, ss)


def _k5b_body(mk_ref, y1_ref, x1_ref, y2_ref, x2_ref,
              nd_ref, fs_ref, fc_ref, fy1_ref, fx1_ref, fy2_ref, fx2_ref):
    y1 = y1_ref[...]; x1 = x1_ref[...]; y2 = y2_ref[...]; x2 = x2_ref[...]
    masked = mk_ref[...]
    m3 = masked.reshape(B, CP, KW)
    y13 = y1.reshape(B, CP, KW); x13 = x1.reshape(B, CP, KW)
    y23 = y2.reshape(B, CP, KW); x23 = x2.reshape(B, CP, KW)
    fkey = (lax.broadcasted_iota(jnp.int32, (B, CP, KW), 1) * KW
            + lax.broadcasted_iota(jnp.int32, (B, CP, KW), 2))
    BIGK = jnp.int32(1 << 22)

    def _bb(a):
        return lax.broadcast_in_dim(a, (B, CP, KW), (0,))

    def _bo(a):
        return lax.broadcast_in_dim(a, (B, 128), (0,))

    oit = lax.broadcasted_iota(jnp.int32, (B, 128), 1)
    oz = jnp.zeros((B, 128), jnp.float32)

    def sel_body(t, carry):
        mcur, cnt, os_, oc_, oy1, ox1, oy2, ox2 = carry
        mx = jnp.max(jnp.max(mcur, axis=2), axis=1)
        tie = mcur == _bb(mx)
        key = jnp.where(tie, fkey, BIGK)
        kmin = jnp.min(jnp.min(key, axis=2), axis=1)
        sel = fkey == _bb(kmin)

        def pick(p3):
            return _bo(jnp.sum(jnp.sum(jnp.where(sel, p3, 0.0), axis=2), axis=1))

        slot = oit == t
        os_ = jnp.where(slot, _bo(mx), os_)
        oc_ = jnp.where(slot, _bo(kmin // KW + 1).astype(jnp.float32), oc_)
        oy1 = jnp.where(slot, pick(y13), oy1)
        ox1 = jnp.where(slot, pick(x13), ox1)
        oy2 = jnp.where(slot, pick(y23), oy2)
        ox2 = jnp.where(slot, pick(x23), ox2)
        cnt = cnt + lax.broadcast_in_dim((mx > 0.0).astype(jnp.int32), (B, 1), (0,))
        mcur = jnp.where(sel, NEG_INF, mcur)
        return (mcur, cnt, os_, oc_, oy1, ox1, oy2, ox2)

    _, cnt, os_, oc_, oy1, ox1, oy2, ox2 = lax.fori_loop(
        0, POST, sel_body,
        (m3, jnp.zeros((B, 1), jnp.int32), oz, oz, oz, oz, oz, oz))
    nd_ref[...] = cnt
    fs_ref[...] = os_[:, :POST]
    fc_ref[...] = oc_[:, :POST]
    fy1_ref[...] = oy1[:, :POST]
    fx1_ref[...] = ox1[:, :POST]
    fy2_ref[...] = oy2[:, :POST]
    fx2_ref[...] = ox2[:, :POST]


def _k5b(mk, y1, x1, y2, x2):
    o = jax.ShapeDtypeStruct((B, POST), jnp.float32)
    return pl.pallas_call(
        _k5b_body,
        out_shape=[jax.ShapeDtypeStruct((B, 1), jnp.int32), o, o, o, o, o, o],
    )(mk, y1, x1, y2, x2)


def kernel(class_outputs, box_outputs, anchor_boxes, image_info):
    st = _k1a(class_outputs)
    tbits, ngt = _k1b(st)
    st = st.reshape(R, N)
    tb16 = jnp.broadcast_to(
        lax.bitcast_convert_type(tbits, jnp.float32).reshape(R, 1), (R, 16))
    eq16 = jnp.pad((PRE - ngt).reshape(R, 1), ((0, 0), (0, 15)))
    ss, si = _k2(st, tb16, eq16)
    ss, si = _k3(ss, si)
    boxrows = box_outputs.reshape(B * N * C * 4)
    anchrows = anchor_boxes.reshape(B * N * 4)
    hwh = jnp.broadcast_to(
        jnp.repeat(image_info[:, 0], CP).reshape(R, 1), (R, 16))
    hww = jnp.broadcast_to(
        jnp.repeat(image_info[:, 1], CP).reshape(R, 1), (R, 16))
    y1, x1, y2, x2 = _k4(si, boxrows, anchrows, hwh, hww)
    mk = _k5a(y1, x1, y2, x2, ss)
    nd, fs, fc, fy1, fx1, fy2, fx2 = _k5b(mk, y1, x1, y2, x2)
    final_boxes = jnp.stack([fy1, fx1, fy2, fx2], axis=-1)
    return nd.reshape(B).astype(jnp.int32), final_boxes, fc, fs


# speedup vs baseline: 1.1450x; 1.1450x over previous
import jax
import jax.numpy as jnp
import numpy as np
from jax import lax
from jax.experimental import pallas as pl
from jax.experimental.pallas import tpu as pltpu
from jax.experimental.pallas import tpu_sc as plsc

B, N, C = 2, 20000, 91
CF = C - 1
CP = 96
R = B * CP
KW = 1024
PRE = 1000
POST = 100
NV = N // 16
NW = 32
RPW = R // NW
NB = KW // 128
BBOX_XFORM_CLIP = float(np.log(1000.0 / 16.0))
NEG_INF = float("-inf")


def _k1a_body(cls_ref, st_ref):
    x = cls_ref[0]
    m = jnp.max(x, axis=-1, keepdims=True)
    e = jnp.exp(x - m)
    s = jnp.sum(e, axis=-1, keepdims=True)
    p = e / s
    pf = p[:, 1:]
    pp = jnp.concatenate([pf, jnp.zeros((N, CP - CF), jnp.float32)], axis=1)
    st_ref[0] = pp.T


def _k1a(class_outputs):
    return pl.pallas_call(
        _k1a_body,
        grid=(B,),
        in_specs=[pl.BlockSpec((1, N, C), lambda b: (b, 0, 0))],
        out_specs=pl.BlockSpec((1, CP, N), lambda b: (b, 0, 0)),
        out_shape=jax.ShapeDtypeStruct((B, CP, N), jnp.float32),
    )(class_outputs)


def _k1b_body(st_ref, tb_ref, ngt_ref):
    s = st_ref[0]
    bits = lax.bitcast_convert_type(s, jnp.int32)

    def bs(_, lohi):
        lo, hi = lohi
        mid = lo + (hi - lo) // 2
        cnt = jnp.sum((bits > mid).astype(jnp.int32), axis=1, keepdims=True)
        take = cnt < PRE
        return (jnp.where(take, lo, mid + 1), jnp.where(take, mid, hi))

    lo0 = jnp.zeros((CP, 1), jnp.int32)
    hi0 = jnp.full((CP, 1), jnp.int32(0x3F800000))
    lo, _ = lax.fori_loop(0, 31, bs, (lo0, hi0))
    ngt = jnp.sum((bits > lo).astype(jnp.int32), axis=1, keepdims=True)
    tb_ref[0] = lo
    ngt_ref[0] = ngt


def _k1b(st):
    return pl.pallas_call(
        _k1b_body,
        grid=(B,),
        in_specs=[pl.BlockSpec((1, CP, N), lambda b: (b, 0, 0))],
        out_specs=[
            pl.BlockSpec((1, CP, 1), lambda b: (b, 0, 0)),
            pl.BlockSpec((1, CP, 1), lambda b: (b, 0, 0)),
        ],
        out_shape=[
            jax.ShapeDtypeStruct((B, CP, 1), jnp.int32),
            jax.ShapeDtypeStruct((B, CP, 1), jnp.int32),
        ],
    )(st)


def _k2_body(st_hbm, tb_hbm, eq_hbm, ss_hbm, si_hbm,
             row_v, sbuf, ibuf, ebuf, t_v, e_v, sem):
    core = lax.axis_index("c")
    sub = lax.axis_index("s")
    wid = sub * 2 + core
    for rr in range(RPW):
        r = wid * RPW + rr
        pltpu.sync_copy(st_hbm.at[r], row_v)
        pltpu.sync_copy(tb_hbm.at[r], t_v)
        pltpu.sync_copy(eq_hbm.at[r], e_v)
        tvec = t_v[...]
        eq0 = jnp.sum(e_v[...], axis=0)
        for t in range(62, KW // 16 + 1):
            sbuf[pl.ds(t * 16, 16)] = jnp.full((16,), NEG_INF, jnp.float32)
            ibuf[pl.ds(t * 16, 16)] = jnp.zeros((16,), jnp.int32)

        def body(jj, carry):
            ptr, eptr = carry
            sv = row_v[pl.ds(jj * 16, 16)]
            gt = sv > tvec
            eq = sv == tvec
            cnt = jnp.sum(gt.astype(jnp.int32), axis=0)
            ecnt = jnp.sum(eq.astype(jnp.int32), axis=0)
            idxv = lax.iota(jnp.int32, 16) + jj * 16
            plsc.store_compressed(sbuf.at[pl.ds(ptr, 16)], sv, mask=gt)
            plsc.store_compressed(ibuf.at[pl.ds(ptr, 16)], idxv, mask=gt)
            plsc.store_compressed(ebuf.at[pl.ds(eptr, 16)], idxv, mask=eq)
            return ptr + cnt, eptr + ecnt

        ngt, _ = lax.fori_loop(0, NV, body, (jnp.int32(0), jnp.int32(0)))

        def post(t, carry):
            base = t * 16
            lm = (lax.iota(jnp.int32, 16) + base) < eq0
            ev = ebuf[pl.ds(base, 16)]
            plsc.store_compressed(sbuf.at[pl.ds(ngt + base, 16)], tvec, mask=lm)
            plsc.store_compressed(ibuf.at[pl.ds(ngt + base, 16)], ev, mask=lm)
            return carry

        lax.fori_loop(0, (eq0 + 15) // 16, post, 0)
        pltpu.sync_copy(sbuf.at[pl.ds(0, KW)], ss_hbm.at[r])
        pltpu.sync_copy(ibuf.at[pl.ds(0, KW)], si_hbm.at[r])


def _k2(st, tb16, eq16):
    mesh = plsc.VectorSubcoreMesh(core_axis_name="c", subcore_axis_name="s")
    return pl.kernel(
        _k2_body,
        out_type=[
            jax.ShapeDtypeStruct((R, KW), jnp.float32),
            jax.ShapeDtypeStruct((R, KW), jnp.int32),
        ],
        mesh=mesh,
        compiler_params=pltpu.CompilerParams(needs_layout_passes=False),
        scratch_types=[
            pltpu.VMEM((N,), jnp.float32),
            pltpu.VMEM((KW + 16,), jnp.float32),
            pltpu.VMEM((KW + 16,), jnp.int32),
            pltpu.VMEM((N + 16,), jnp.int32),
            pltpu.VMEM((16,), jnp.float32),
            pltpu.VMEM((16,), jnp.int32),
            pltpu.SemaphoreType.DMA,
        ],
    )(st, tb16, eq16)


def _rl(x, j):
    return jnp.concatenate([x[:, j:], x[:, :j]], axis=1)


def _rr(x, j):
    return jnp.concatenate([x[:, KW - j:], x[:, :KW - j]], axis=1)


def _k3_body(s_ref, i_ref, so_ref, io_ref):
    s = s_ref[...]
    ii = i_ref[...]
    it = lax.broadcasted_iota(jnp.int32, (R, KW), 1)
    k = 2
    while k <= KW:
        j = k // 2
        while j >= 1:
            bitj = (it & j) != 0
            sp = jnp.where(bitj, _rr(s, j), _rl(s, j))
            ip = jnp.where(bitj, _rr(ii, j), _rl(ii, j))
            first = (s > sp) | ((s == sp) & (ii < ip))
            want_first = (~bitj) == ((it & k) == 0)
            take_self = first == want_first
            s = jnp.where(take_self, s, sp)
            ii = jnp.where(take_self, ii, ip)
            j //= 2
        k *= 2
    so_ref[...] = s
    io_ref[...] = ii


def _k3(s, i):
    return pl.pallas_call(
        _k3_body,
        out_shape=[
            jax.ShapeDtypeStruct((R, KW), jnp.float32),
            jax.ShapeDtypeStruct((R, KW), jnp.int32),
        ],
    )(s, i)


def _k4_body(si_hbm, boxflat_hbm, anchflat_hbm, hwh_hbm, hww_hbm,
             y1_hbm, x1_hbm, y2_hbm, x2_hbm,
             idx_v, eim, aim,
             eyb, exb, ehb, ewb, ay1b, ax1b, ay2b, ax2b,
             y1b, x1b, y2b, x2b, hwh_v, hww_v, sem):
    core = lax.axis_index("c")
    sub = lax.axis_index("s")
    wid = sub * 2 + core
    ebufs = [eyb, exb, ehb, ewb]
    abufs = [ay1b, ax1b, ay2b, ax2b]

    def row_fn(rr, _):
        r = wid * RPW + rr
        b = r // CP
        c = r % CP
        cls = jnp.minimum(c + 1, CF)
        base_e4 = (b * (N * C) + cls) * 4
        base_a4 = b * N * 4
        pltpu.sync_copy(si_hbm.at[r], idx_v)
        pltpu.sync_copy(hwh_hbm.at[r], hwh_v)
        pltpu.sync_copy(hww_hbm.at[r], hww_v)
        hv = hwh_v[...]
        wv = hww_v[...]

        def fill(jj, carry):
            v = idx_v[pl.ds(jj * 16, 16)]
            e4 = v * (C * 4) + base_e4
            a4 = v * 4 + base_a4
            ch = jj // 8
            off = (jj % 8) * 16
            for f in range(4):
                eim[f * 8 + ch, pl.ds(off, 16)] = e4 + f
                aim[f * 8 + ch, pl.ds(off, 16)] = a4 + f
            return carry

        lax.fori_loop(0, KW // 16, fill, 0)
        cps = []
        for f in range(4):
            for ch in range(KW // 128):
                cps.append(pltpu.async_copy(
                    boxflat_hbm.at[eim.at[f * 8 + ch]],
                    ebufs[f].at[pl.ds(ch * 128, 128)], sem))
                cps.append(pltpu.async_copy(
                    anchflat_hbm.at[aim.at[f * 8 + ch]],
                    abufs[f].at[pl.ds(ch * 128, 128)], sem))
        for cp in cps:
            cp.wait()

        def dec(jj, carry):
            sl = pl.ds(jj * 16, 16)
            ey = eyb[sl]
            ex = exb[sl]
            eh = ehb[sl]
            ew = ewb[sl]
            ay1 = ay1b[sl]
            ax1 = ax1b[sl]
            ay2 = ay2b[sl]
            ax2 = ax2b[sl]
            dy = ey / 10.0
            dx = ex / 10.0
            dh = jnp.minimum(eh / 5.0, BBOX_XFORM_CLIP)
            dw = jnp.minimum(ew / 5.0, BBOX_XFORM_CLIP)
            a_h = ay2 - ay1
            a_w = ax2 - ax1
            a_cy = ay1 + 0.5 * a_h
            a_cx = ax1 + 0.5 * a_w
            cy = dy * a_h + a_cy
            cx = dx * a_w + a_cx
            hh = jnp.exp(dh) * a_h
            ww = jnp.exp(dw) * a_w
            zf = jnp.zeros((16,), jnp.float32)
            y1b[sl] = jnp.minimum(jnp.maximum(cy - 0.5 * hh, zf), hv)
            x1b[sl] = jnp.minimum(jnp.maximum(cx - 0.5 * ww, zf), wv)
            y2b[sl] = jnp.minimum(jnp.maximum(cy + 0.5 * hh, zf), hv)
            x2b[sl] = jnp.minimum(jnp.maximum(cx + 0.5 * ww, zf), wv)
            return carry

        lax.fori_loop(0, KW // 16, dec, 0)
        pltpu.sync_copy(y1b, y1_hbm.at[r])
        pltpu.sync_copy(x1b, x1_hbm.at[r])
        pltpu.sync_copy(y2b, y2_hbm.at[r])
        pltpu.sync_copy(x2b, x2_hbm.at[r])
        return _

    lax.fori_loop(0, RPW, row_fn, 0)


def _k4(si, boxrows, anchrows, hwh, hww):
    mesh = plsc.VectorSubcoreMesh(core_axis_name="c", subcore_axis_name="s")
    plane = jax.ShapeDtypeStruct((R, KW), jnp.float32)
    return pl.kernel(
        _k4_body,
        out_type=[plane, plane, plane, plane],
        mesh=mesh,
        compiler_params=pltpu.CompilerParams(needs_layout_passes=False),
        scratch_types=(
            [pltpu.VMEM((KW,), jnp.int32),
             pltpu.VMEM((32, 128), jnp.int32),
             pltpu.VMEM((32, 128), jnp.int32)]
            + [pltpu.VMEM((KW,), jnp.float32) for _ in range(12)]
            + [pltpu.VMEM((16,), jnp.float32) for _ in range(2)]
            + [pltpu.SemaphoreType.DMA]
        ),
    )(si, boxrows, anchrows, hwh, hww)


RB = 16


def _k5a_body(y1_ref, x1_ref, y2_ref, x2_ref, ss_ref, mk_ref):
    y1 = y1_ref[...]; x1 = x1_ref[...]; y2 = y2_ref[...]; x2 = x2_ref[...]
    area = (y2 - y1) * (x2 - x1)
    kb = [jnp.ones((RB, 128), jnp.bool_) for _ in range(NB)]

    def _bi(a):
        return lax.broadcast_in_dim(a, (RB, 128, 128), (0, 1))

    def _bj(a):
        return lax.broadcast_in_dim(a, (RB, 128, 128), (0, 2))

    def iou_blk(ai, bj):
        sa = slice(ai * 128, (ai + 1) * 128)
        sb = slice(bj * 128, (bj + 1) * 128)
        ih = jnp.maximum(
            jnp.minimum(_bi(y2[:, sa]), _bj(y2[:, sb]))
            - jnp.maximum(_bi(y1[:, sa]), _bj(y1[:, sb])), 0.0)
        iw = jnp.maximum(
            jnp.minimum(_bi(x2[:, sa]), _bj(x2[:, sb]))
            - jnp.maximum(_bi(x1[:, sa]), _bj(x1[:, sb])), 0.0)
        inter = ih * iw
        denom = _bi(area[:, sa]) + _bj(area[:, sb]) - inter + 1e-8
        return inter / denom > 0.5

    tri = (lax.broadcasted_iota(jnp.int32, (1, 128, 128), 1)
           < lax.broadcasted_iota(jnp.int32, (1, 128, 128), 2))
    for bi in range(NB):
        M = iou_blk(bi, bi) & tri
        pre = kb[bi]

        def _bim(a_i32):
            return _bi(a_i32) != 0

        def fx_body(carry):
            alive_i, _ = carry
            supp = jnp.any(M & _bim(alive_i), axis=1)
            new = pre & (~supp)
            new_i = new.astype(jnp.int32)
            return (new_i, jnp.any(new_i != alive_i))

        alive_i, _ = lax.while_loop(
            lambda cr: cr[1], fx_body, (pre.astype(jnp.int32), jnp.bool_(True)))
        alive = alive_i != 0
        kb[bi] = alive
        for bj in range(bi + 1, NB):
            Mc = iou_blk(bi, bj)
            supp = jnp.any(Mc & _bim(alive_i), axis=1)
            kb[bj] = kb[bj] & (~supp)

    keep = jnp.concatenate(kb, axis=1)
    ss = ss_ref[...]
    masked = jnp.where(keep, ss, -1.0)
    colit = lax.broadcasted_iota(jnp.int32, (RB, KW), 1)
    rowit = (lax.broadcasted_iota(jnp.int32, (RB, KW), 0)
             + pl.program_id(0) * RB)
    valid = (colit < PRE) & ((rowit % CP) < CF)
    mk_ref[...] = jnp.where(valid, masked, NEG_INF)


def _k5a(y1, x1, y2, x2, ss):
    bs = pl.BlockSpec((RB, KW), lambda g: (g, 0))
    return pl.pallas_call(
        _k5a_body,
        grid=(R // RB,),
        in_specs=[bs, bs, bs, bs, bs],
        out_specs=bs,
        out_shape=jax.ShapeDtypeStruct((R, KW), jnp.float32),
    )(y1, x1, y2, x2, ss)


def _k5b_body(mk_ref, y1_ref, x1_ref, y2_ref, x2_ref,
              nd_ref, fs_ref, fc_ref, fy1_ref, fx1_ref, fy2_ref, fx2_ref):
    y1 = y1_ref[...]; x1 = x1_ref[...]; y2 = y2_ref[...]; x2 = x2_ref[...]
    masked = mk_ref[...]
    m3 = masked.reshape(B, CP, KW)
    y13 = y1.reshape(B, CP, KW); x13 = x1.reshape(B, CP, KW)
    y23 = y2.reshape(B, CP, KW); x23 = x2.reshape(B, CP, KW)
    fkey = (lax.broadcasted_iota(jnp.int32, (B, CP, KW), 1) * KW
            + lax.broadcasted_iota(jnp.int32, (B, CP, KW), 2))
    BIGK = jnp.int32(1 << 22)

    def _bb(a):
        return lax.broadcast_in_dim(a, (B, CP, KW), (0,))

    def _bo(a):
        return lax.broadcast_in_dim(a, (B, 128), (0,))

    oit = lax.broadcasted_iota(jnp.int32, (B, 128), 1)
    oz = jnp.zeros((B, 128), jnp.float32)

    def sel_body(t, carry):
        mcur, cnt, os_, oc_, oy1, ox1, oy2, ox2 = carry
        mx = jnp.max(jnp.max(mcur, axis=2), axis=1)
        tie = mcur == _bb(mx)
        key = jnp.where(tie, fkey, BIGK)
        kmin = jnp.min(jnp.min(key, axis=2), axis=1)
        sel = fkey == _bb(kmin)

        def pick(p3):
            return _bo(jnp.sum(jnp.sum(jnp.where(sel, p3, 0.0), axis=2), axis=1))

        slot = oit == t
        os_ = jnp.where(slot, _bo(mx), os_)
        oc_ = jnp.where(slot, _bo(kmin // KW + 1).astype(jnp.float32), oc_)
        oy1 = jnp.where(slot, pick(y13), oy1)
        ox1 = jnp.where(slot, pick(x13), ox1)
        oy2 = jnp.where(slot, pick(y23), oy2)
        ox2 = jnp.where(slot, pick(x23), ox2)
        cnt = cnt + lax.broadcast_in_dim((mx > 0.0).astype(jnp.int32), (B, 1), (0,))
        mcur = jnp.where(sel, NEG_INF, mcur)
        return (mcur, cnt, os_, oc_, oy1, ox1, oy2, ox2)

    _, cnt, os_, oc_, oy1, ox1, oy2, ox2 = lax.fori_loop(
        0, POST, sel_body,
        (m3, jnp.zeros((B, 1), jnp.int32), oz, oz, oz, oz, oz, oz))
    nd_ref[...] = cnt
    fs_ref[...] = os_[:, :POST]
    fc_ref[...] = oc_[:, :POST]
    fy1_ref[...] = oy1[:, :POST]
    fx1_ref[...] = ox1[:, :POST]
    fy2_ref[...] = oy2[:, :POST]
    fx2_ref[...] = ox2[:, :POST]


def _k5b(mk, y1, x1, y2, x2):
    o = jax.ShapeDtypeStruct((B, POST), jnp.float32)
    return pl.pallas_call(
        _k5b_body,
        out_shape=[jax.ShapeDtypeStruct((B, 1), jnp.int32), o, o, o, o, o, o],
    )(mk, y1, x1, y2, x2)


def kernel(class_outputs, box_outputs, anchor_boxes, image_info):
    st = _k1a(class_outputs)
    tbits, ngt = _k1b(st)
    st = st.reshape(R, N)
    tb16 = jnp.broadcast_to(
        lax.bitcast_convert_type(tbits, jnp.float32).reshape(R, 1), (R, 16))
    eq16 = jnp.pad((PRE - ngt).reshape(R, 1), ((0, 0), (0, 15)))
    ss, si = _k2(st, tb16, eq16)
    ss, si = _k3(ss, si)
    boxrows = box_outputs.reshape(B * N * C * 4)
    anchrows = anchor_boxes.reshape(B * N * 4)
    hwh = jnp.broadcast_to(
        jnp.repeat(image_info[:, 0], CP).reshape(R, 1), (R, 16))
    hww = jnp.broadcast_to(
        jnp.repeat(image_info[:, 1], CP).reshape(R, 1), (R, 16))
    y1, x1, y2, x2 = _k4(si, boxrows, anchrows, hwh, hww)
    mk = _k5a(y1, x1, y2, x2, ss)
    nd, fs, fc, fy1, fx1, fy2, fx2 = _k5b(mk, y1, x1, y2, x2)
    final_boxes = jnp.stack([fy1, fx1, fy2, fx2], axis=-1)
    return nd.reshape(B).astype(jnp.int32), final_boxes, fc, fs
